# baseline (speedup 1.0000x reference)
import functools

import jax
import jax.numpy as jnp
from jax import lax
from jax.experimental import pallas as pl
from jax.experimental.pallas import tpu as pltpu
from jax.experimental.pallas import tpu_sc as plsc

_T = 256
_CA = 256
_NW = 32


def _gate_body(nE, T, nt, x_ref, gw_ref, gb_ref, w01_ref, e01_ref, base_ref,
               meta_ref, w01w_ref):
    x = x_ref[...]
    N = x.shape[0]
    logits = jax.lax.dot(x, gw_ref[...],
                         preferred_element_type=jnp.float32) + gb_ref[...]
    p = jax.nn.softmax(logits, axis=-1)
    idx = jax.lax.broadcasted_iota(jnp.int32, p.shape, 1)
    m0 = jnp.max(p, axis=-1, keepdims=True)
    e0 = jnp.min(jnp.where(p == m0, idx, nE), axis=-1, keepdims=True)
    p2 = jnp.where(idx == e0, -jnp.inf, p)
    m1 = jnp.max(p2, axis=-1, keepdims=True)
    e1 = jnp.min(jnp.where(p2 == m1, idx, nE), axis=-1, keepdims=True)
    w01_ref[0, :] = m0[:, 0]
    w01_ref[1, :] = m1[:, 0]
    e01_ref[0, :] = e0[:, 0]
    e01_ref[1, :] = e1[:, 0]
    w01w_ref[...] = jnp.concatenate(
        [jnp.broadcast_to(m0, (N, 128)), jnp.broadcast_to(m1, (N, 128))],
        axis=0)

    oh0 = (idx == e0).astype(jnp.float32)
    oh1 = (idx == e1).astype(jnp.float32)
    nch = N // _CA
    ci = jax.lax.broadcasted_iota(jnp.int32, (nch, N), 0)
    tj = jax.lax.broadcasted_iota(jnp.int32, (nch, N), 1) // _CA
    chunk_ind = (ci == tj).astype(jnp.float32)
    cnt0 = jax.lax.dot(chunk_ind, oh0, preferred_element_type=jnp.float32)
    cnt1 = jax.lax.dot(chunk_ind, oh1, preferred_element_type=jnp.float32)
    cnt = jnp.concatenate([cnt0, cnt1], axis=0)
    tot = jnp.sum(cnt, axis=0, keepdims=True)
    padded = jnp.floor((tot + (T - 1)) * (1.0 / T)) * T
    ri = jax.lax.broadcasted_iota(jnp.int32, (2 * nch, 2 * nch), 0)
    rj = jax.lax.broadcasted_iota(jnp.int32, (2 * nch, 2 * nch), 1)
    tril = (ri > rj).astype(jnp.float32)
    prefix = jax.lax.dot(tril, cnt, preferred_element_type=jnp.float32)
    ui = jax.lax.broadcasted_iota(jnp.int32, (nE, nE), 0)
    uj = jax.lax.broadcasted_iota(jnp.int32, (nE, nE), 1)
    sel = (ui < uj).astype(jnp.float32)
    seg_start = jax.lax.dot(padded, sel, preferred_element_type=jnp.float32)
    base = seg_start + prefix
    base_ref[...] = jnp.concatenate(
        [base.astype(jnp.int32),
         jnp.zeros((2 * nch, 16 - nE), jnp.int32)], axis=1)
    ends = seg_start + padded
    lane = jax.lax.broadcasted_iota(jnp.int32, (1, 64), 1)
    ts = (lane * T).astype(jnp.float32)
    acc = jnp.zeros((1, 64), jnp.float32)
    for e in range(nE):
        acc = acc + (ts >= ends[:, e:e + 1]).astype(jnp.float32)
    ept = jnp.minimum(acc, nE - 1)
    nreal = ends[:, nE - 1:nE] * (1.0 / T)
    meta = jnp.where(lane == 40, nreal, jnp.where(lane < nt, ept, 0.0))
    meta_ref[...] = meta.astype(jnp.int32)


def _gate(xf, gate_W, gate_b, T, nt):
    N, D = xf.shape
    E = gate_W.shape[1]
    return pl.pallas_call(
        functools.partial(_gate_body, E, T, nt),
        in_specs=[
            pl.BlockSpec((N, D), lambda: (0, 0)),
            pl.BlockSpec((D, E), lambda: (0, 0)),
            pl.BlockSpec((1, E), lambda: (0, 0)),
        ],
        out_specs=[
            pl.BlockSpec((2, N), lambda: (0, 0)),
            pl.BlockSpec((2, N), lambda: (0, 0)),
            pl.BlockSpec((2 * (N // _CA), 16), lambda: (0, 0)),
            pl.BlockSpec((1, 64), lambda: (0, 0)),
            pl.BlockSpec((2 * N, 128), lambda: (0, 0)),
        ],
        out_shape=[
            jax.ShapeDtypeStruct((2, N), jnp.float32),
            jax.ShapeDtypeStruct((2, N), jnp.int32),
            jax.ShapeDtypeStruct((2 * (N // _CA), 16), jnp.int32),
            jax.ShapeDtypeStruct((1, 64), jnp.int32),
            jax.ShapeDtypeStruct((2 * N, 128), jnp.float32),
        ],
    )(xf, gate_W, gate_b.reshape(1, E))


def _lane_gather(vec, idx):
    dnums = lax.GatherDimensionNumbers(
        offset_dims=(), collapsed_slice_dims=(0,), start_index_map=(0,))
    return lax.gather(vec, idx[:, None], dnums, slice_sizes=(1,),
                      mode=lax.GatherScatterMode.PROMISE_IN_BOUNDS)


def _make_dispatch(N, D, E, Amax):
    mesh = plsc.VectorSubcoreMesh(core_axis_name="c", subcore_axis_name="s")

    @functools.partial(
        pl.kernel, mesh=mesh,
        out_type=[
            jax.ShapeDtypeStruct((Amax, D), jnp.float32),
            jax.ShapeDtypeStruct((_NW, 2, _CA // 2), jnp.int32),
            jax.ShapeDtypeStruct((Amax, 128), jnp.float32),
        ],
        scratch_types=[
            pltpu.VMEM((_CA,), jnp.int32),
            pltpu.VMEM((16,), jnp.int32),
            pltpu.VMEM((2, _CA // 2), jnp.int32),
            pltpu.VMEM((_CA // 2, D), jnp.float32),
            pltpu.VMEM((_CA // 2, 128), jnp.float32),
            pltpu.SemaphoreType.DMA,
        ],
    )
    def dispatch(e01f_hbm, base_hbm, xf_hbm, w01w_hbm, xs_hbm, dst_hbm,
                 sc_hbm, ebuf, runbuf, dstbuf, xbuf, swbuf, sem):
        w = lax.axis_index("s") * 2 + lax.axis_index("c")
        tok0 = (w % 16) * _CA
        half_rows = _CA // 2
        pltpu.sync_copy(e01f_hbm.at[pl.ds(w * _CA, _CA)], ebuf)
        pltpu.sync_copy(base_hbm.at[w], runbuf)

        iota16 = jax.lax.iota(jnp.int32, 16)
        lane15 = jnp.full((16,), 15, jnp.int32)
        shift_idx = [jnp.maximum(iota16 - (1 << s), 0) for s in range(4)]
        run = runbuf[...]
        for v in range(_CA // 16):
            ae = ebuf[pl.ds(v * 16, 16)]
            dstv = jnp.zeros((16,), jnp.int32)
            for e in range(E):
                mask = ae == e
                csum = jnp.where(mask, 1, 0)
                for s in range(4):
                    moved = _lane_gather(csum, shift_idx[s])
                    csum = csum + jnp.where(iota16 >= (1 << s), moved, 0)
                base_e = _lane_gather(run, jnp.full((16,), e, jnp.int32))
                dstv = jnp.where(mask, base_e + csum - 1, dstv)
                tot = _lane_gather(csum, lane15)
                run = run + jnp.where(iota16 == e, tot, 0)
            dstbuf[v // 8, pl.ds((v % 8) * 16, 16)] = dstv
        pltpu.sync_copy(dstbuf, dst_hbm.at[w])
        for half in range(2):
            pltpu.sync_copy(
                xf_hbm.at[pl.ds(tok0 + half * half_rows, half_rows)], xbuf)
            pltpu.async_copy(xbuf, xs_hbm.at[dstbuf.at[half]], sem).wait()
            pltpu.sync_copy(
                w01w_hbm.at[pl.ds(w * _CA + half * half_rows, half_rows)],
                swbuf)
            pltpu.async_copy(swbuf, sc_hbm.at[dstbuf.at[half]], sem).wait()

    return dispatch


def _ffn_body(ept_ref, nreal_ref, x_ref, w1_ref, b1_ref, w2_ref, b2_ref,
              s_ref, y_ref):
    i = pl.program_id(0)
    f = pl.program_id(1)

    @pl.when(i < nreal_ref[0])
    def _():
        x = x_ref[...]
        h = jax.lax.dot(x, w1_ref[0], preferred_element_type=jnp.float32)
        h = h + b1_ref[0]
        h = 0.5 * h * (1.0 + jax.lax.erf(h * (2.0 ** -0.5)))
        part = jax.lax.dot(h, w2_ref[0], preferred_element_type=jnp.float32)

        @pl.when(f == 0)
        def _():
            y_ref[...] = part

        @pl.when(f == 1)
        def _():
            scale = jnp.max(s_ref[...], axis=1, keepdims=True)
            y_ref[...] = (y_ref[...] + part + b2_ref[0]) * scale


def _grouped_ffn(x_sorted, ept, nreal, W1, b1, W2, b2, scores, T):
    Amax, D = x_sorted.shape
    E, _, F = W1.shape
    Fh = F // 2
    nt = Amax // T
    grid_spec = pltpu.PrefetchScalarGridSpec(
        num_scalar_prefetch=2,
        grid=(nt, 2),
        in_specs=[
            pl.BlockSpec((T, D), lambda i, f, ept, nr: (i, 0)),
            pl.BlockSpec((1, D, Fh), lambda i, f, ept, nr: (ept[i], 0, f)),
            pl.BlockSpec((1, 1, Fh), lambda i, f, ept, nr: (ept[i], 0, f)),
            pl.BlockSpec((1, Fh, D), lambda i, f, ept, nr: (ept[i], f, 0)),
            pl.BlockSpec((1, 1, D), lambda i, f, ept, nr: (ept[i], 0, 0)),
            pl.BlockSpec((T, 128), lambda i, f, ept, nr: (i, 0)),
        ],
        out_specs=pl.BlockSpec((T, D), lambda i, f, ept, nr: (i, 0)),
    )
    return pl.pallas_call(
        _ffn_body,
        grid_spec=grid_spec,
        out_shape=jax.ShapeDtypeStruct((Amax, D), jnp.float32),
        compiler_params=pltpu.CompilerParams(
            dimension_semantics=("arbitrary", "arbitrary"),
        ),
    )(ept, nreal, x_sorted, W1, b1.reshape(E, 1, F), W2, b2.reshape(E, 1, D),
      scores)


def _make_combine(N, D, Amax):
    mesh = plsc.VectorSubcoreMesh(core_axis_name="c", subcore_axis_name="s")
    tw = N // _NW
    half_rows = tw // 2

    @functools.partial(
        pl.kernel, mesh=mesh,
        out_type=jax.ShapeDtypeStruct((N, D), jnp.float32),
        scratch_types=[
            pltpu.VMEM((2, tw), jnp.int32),
            pltpu.VMEM((half_rows, D), jnp.float32),
            pltpu.VMEM((half_rows, D), jnp.float32),
            pltpu.SemaphoreType.DMA,
        ],
    )
    def combine(y_hbm, dst_hbm, out_hbm, idxbuf, ybuf0, ybuf1, sem):
        w = lax.axis_index("s") * 2 + lax.axis_index("c")
        nch = _NW // 2
        pltpu.sync_copy(dst_hbm.at[w // 2, w % 2], idxbuf.at[0])
        pltpu.sync_copy(dst_hbm.at[nch + w // 2, w % 2], idxbuf.at[1])
        nd = D // 16
        for half in range(2):
            i0 = idxbuf.at[0, pl.ds(half * half_rows, half_rows)]
            i1 = idxbuf.at[1, pl.ds(half * half_rows, half_rows)]
            pltpu.async_copy(y_hbm.at[i0], ybuf0, sem).wait()
            pltpu.async_copy(y_hbm.at[i1], ybuf1, sem).wait()

            def row(r, carry):
                for d in range(nd):
                    sl = pl.ds(d * 16, 16)
                    ybuf0[r, sl] = ybuf0[r, sl] + ybuf1[r, sl]
                return carry

            lax.fori_loop(0, half_rows, row, 0)
            pltpu.sync_copy(
                ybuf0,
                out_hbm.at[pl.ds(w * tw + half * half_rows, half_rows)])

    return combine


def kernel(x, gate_W, gate_b, W1, b1, W2, b2):
    B, L, D = x.shape
    E = gate_W.shape[1]
    N = B * L
    A = 2 * N
    T = _T
    Amax = A + 8 * T
    nt = Amax // T

    xf = x.reshape(N, D)
    w01, e01, base, meta, w01w = _gate(xf, gate_W, gate_b, T, nt)
    ept = meta[0, :nt]
    nreal = meta[0, 40:41]

    x_sorted, dst, scores = _make_dispatch(N, D, E, Amax)(
        e01.reshape(A), base, xf, w01w)

    y_sorted = _grouped_ffn(x_sorted, ept, nreal, W1, b1, W2, b2, scores, T)

    out = _make_combine(N, D, Amax)(y_sorted, dst)
    return out.reshape(B, L, D)


# speedup vs baseline: 1.4132x; 1.4132x over previous
import functools

import jax
import jax.numpy as jnp
from jax import lax
from jax.experimental import pallas as pl
from jax.experimental.pallas import tpu as pltpu
from jax.experimental.pallas import tpu_sc as plsc

_T = 256
_CA = 256
_NW = 32


def _gate_body(nE, T, nt, x_ref, gw_ref, gb_ref, w01_ref, e01_ref, base_ref,
               meta_ref, w01w_ref):
    x = x_ref[...]
    N = x.shape[0]
    logits = jax.lax.dot(x, gw_ref[...],
                         preferred_element_type=jnp.float32) + gb_ref[...]
    p = jax.nn.softmax(logits, axis=-1)
    idx = jax.lax.broadcasted_iota(jnp.int32, p.shape, 1)
    m0 = jnp.max(p, axis=-1, keepdims=True)
    e0 = jnp.min(jnp.where(p == m0, idx, nE), axis=-1, keepdims=True)
    p2 = jnp.where(idx == e0, -jnp.inf, p)
    m1 = jnp.max(p2, axis=-1, keepdims=True)
    e1 = jnp.min(jnp.where(p2 == m1, idx, nE), axis=-1, keepdims=True)
    w01_ref[0, :] = m0[:, 0]
    w01_ref[1, :] = m1[:, 0]
    e01_ref[0, :] = e0[:, 0]
    e01_ref[1, :] = e1[:, 0]
    w01w_ref[...] = jnp.concatenate(
        [jnp.broadcast_to(m0, (N, 128)), jnp.broadcast_to(m1, (N, 128))],
        axis=0)

    oh0 = (idx == e0).astype(jnp.float32)
    oh1 = (idx == e1).astype(jnp.float32)
    nch = N // _CA
    ci = jax.lax.broadcasted_iota(jnp.int32, (nch, N), 0)
    tj = jax.lax.broadcasted_iota(jnp.int32, (nch, N), 1) // _CA
    chunk_ind = (ci == tj).astype(jnp.float32)
    cnt0 = jax.lax.dot(chunk_ind, oh0, preferred_element_type=jnp.float32)
    cnt1 = jax.lax.dot(chunk_ind, oh1, preferred_element_type=jnp.float32)
    cnt = jnp.concatenate([cnt0, cnt1], axis=0)
    tot = jnp.sum(cnt, axis=0, keepdims=True)
    padded = jnp.floor((tot + (T - 1)) * (1.0 / T)) * T
    ri = jax.lax.broadcasted_iota(jnp.int32, (2 * nch, 2 * nch), 0)
    rj = jax.lax.broadcasted_iota(jnp.int32, (2 * nch, 2 * nch), 1)
    tril = (ri > rj).astype(jnp.float32)
    prefix = jax.lax.dot(tril, cnt, preferred_element_type=jnp.float32)
    ui = jax.lax.broadcasted_iota(jnp.int32, (nE, nE), 0)
    uj = jax.lax.broadcasted_iota(jnp.int32, (nE, nE), 1)
    sel = (ui < uj).astype(jnp.float32)
    seg_start = jax.lax.dot(padded, sel, preferred_element_type=jnp.float32)
    base = seg_start + prefix
    base_ref[...] = jnp.concatenate(
        [base.astype(jnp.int32),
         jnp.zeros((2 * nch, 16 - nE), jnp.int32)], axis=1)
    ends = seg_start + padded
    lane = jax.lax.broadcasted_iota(jnp.int32, (1, 64), 1)
    ts = (lane * T).astype(jnp.float32)
    acc = jnp.zeros((1, 64), jnp.float32)
    for e in range(nE):
        acc = acc + (ts >= ends[:, e:e + 1]).astype(jnp.float32)
    ept = jnp.minimum(acc, nE - 1)
    nreal = ends[:, nE - 1:nE] * (1.0 / T)
    meta = jnp.where(lane == 40, nreal, jnp.where(lane < nt, ept, 0.0))
    meta_ref[...] = meta.astype(jnp.int32)


def _gate(xf, gate_W, gate_b, T, nt):
    N, D = xf.shape
    E = gate_W.shape[1]
    return pl.pallas_call(
        functools.partial(_gate_body, E, T, nt),
        in_specs=[
            pl.BlockSpec((N, D), lambda: (0, 0)),
            pl.BlockSpec((D, E), lambda: (0, 0)),
            pl.BlockSpec((1, E), lambda: (0, 0)),
        ],
        out_specs=[
            pl.BlockSpec((2, N), lambda: (0, 0)),
            pl.BlockSpec((2, N), lambda: (0, 0)),
            pl.BlockSpec((2 * (N // _CA), 16), lambda: (0, 0)),
            pl.BlockSpec((1, 64), lambda: (0, 0)),
            pl.BlockSpec((2 * N, 128), lambda: (0, 0)),
        ],
        out_shape=[
            jax.ShapeDtypeStruct((2, N), jnp.float32),
            jax.ShapeDtypeStruct((2, N), jnp.int32),
            jax.ShapeDtypeStruct((2 * (N // _CA), 16), jnp.int32),
            jax.ShapeDtypeStruct((1, 64), jnp.int32),
            jax.ShapeDtypeStruct((2 * N, 128), jnp.float32),
        ],
    )(xf, gate_W, gate_b.reshape(1, E))


def _lane_gather(vec, idx):
    dnums = lax.GatherDimensionNumbers(
        offset_dims=(), collapsed_slice_dims=(0,), start_index_map=(0,))
    return lax.gather(vec, idx[:, None], dnums, slice_sizes=(1,),
                      mode=lax.GatherScatterMode.PROMISE_IN_BOUNDS)


def _make_dispatch(N, D, E, Amax):
    mesh = plsc.VectorSubcoreMesh(core_axis_name="c", subcore_axis_name="s")

    @functools.partial(
        pl.kernel, mesh=mesh,
        out_type=[
            jax.ShapeDtypeStruct((Amax, D), jnp.float32),
            jax.ShapeDtypeStruct((_NW, 2, _CA // 2), jnp.int32),
            jax.ShapeDtypeStruct((Amax, 128), jnp.float32),
        ],
        scratch_types=[
            pltpu.VMEM((_CA,), jnp.int32),
            pltpu.VMEM((16,), jnp.int32),
            pltpu.VMEM((2, _CA // 2), jnp.int32),
            pltpu.VMEM((_CA // 2, D), jnp.float32),
            pltpu.VMEM((_CA // 2, 128), jnp.float32),
            pltpu.SemaphoreType.DMA,
        ],
    )
    def dispatch(e01f_hbm, base_hbm, xf_hbm, w01w_hbm, xs_hbm, dst_hbm,
                 sc_hbm, ebuf, runbuf, dstbuf, xbuf, swbuf, sem):
        w = lax.axis_index("s") * 2 + lax.axis_index("c")
        tok0 = (w % 16) * _CA
        half_rows = _CA // 2
        pltpu.sync_copy(e01f_hbm.at[pl.ds(w * _CA, _CA)], ebuf)
        pltpu.sync_copy(base_hbm.at[w], runbuf)

        iota16 = jax.lax.iota(jnp.int32, 16)
        lane15 = jnp.full((16,), 15, jnp.int32)
        shift_idx = [jnp.maximum(iota16 - (1 << s), 0) for s in range(4)]
        run = runbuf[...]
        for v in range(_CA // 16):
            ae = ebuf[pl.ds(v * 16, 16)]
            dstv = jnp.zeros((16,), jnp.int32)
            for e in range(E):
                mask = ae == e
                csum = jnp.where(mask, 1, 0)
                for s in range(4):
                    moved = _lane_gather(csum, shift_idx[s])
                    csum = csum + jnp.where(iota16 >= (1 << s), moved, 0)
                base_e = _lane_gather(run, jnp.full((16,), e, jnp.int32))
                dstv = jnp.where(mask, base_e + csum - 1, dstv)
                tot = _lane_gather(csum, lane15)
                run = run + jnp.where(iota16 == e, tot, 0)
            dstbuf[v // 8, pl.ds((v % 8) * 16, 16)] = dstv
        pltpu.sync_copy(dstbuf, dst_hbm.at[w])
        for half in range(2):
            pltpu.sync_copy(
                xf_hbm.at[pl.ds(tok0 + half * half_rows, half_rows)], xbuf)
            pltpu.async_copy(xbuf, xs_hbm.at[dstbuf.at[half]], sem).wait()
            pltpu.sync_copy(
                w01w_hbm.at[pl.ds(w * _CA + half * half_rows, half_rows)],
                swbuf)
            pltpu.async_copy(swbuf, sc_hbm.at[dstbuf.at[half]], sem).wait()

    return dispatch


def _ffn_body(ept_ref, nreal_ref, x_ref, w1_ref, b1_ref, w2_ref, b2_ref,
              s_ref, y_ref):
    i = pl.program_id(0)

    @pl.when(i < nreal_ref[0])
    def _():
        x = x_ref[...]
        h = jax.lax.dot(x, w1_ref[0], preferred_element_type=jnp.float32)
        h = h + b1_ref[0]
        h = 0.5 * h * (1.0 + jax.lax.erf(h * (2.0 ** -0.5)))
        y = jax.lax.dot(h, w2_ref[0], preferred_element_type=jnp.float32)
        scale = jnp.max(s_ref[...], axis=1, keepdims=True)
        y_ref[...] = (y + b2_ref[0]) * scale


def _grouped_ffn(x_sorted, ept, nreal, W1, b1, W2, b2, scores, T):
    Amax, D = x_sorted.shape
    E, _, F = W1.shape
    nt = Amax // T
    grid_spec = pltpu.PrefetchScalarGridSpec(
        num_scalar_prefetch=2,
        grid=(nt,),
        in_specs=[
            pl.BlockSpec((T, D), lambda i, ept, nr: (i, 0)),
            pl.BlockSpec((1, D, F), lambda i, ept, nr: (ept[i], 0, 0)),
            pl.BlockSpec((1, 1, F), lambda i, ept, nr: (ept[i], 0, 0)),
            pl.BlockSpec((1, F, D), lambda i, ept, nr: (ept[i], 0, 0)),
            pl.BlockSpec((1, 1, D), lambda i, ept, nr: (ept[i], 0, 0)),
            pl.BlockSpec((T, 128), lambda i, ept, nr: (i, 0)),
        ],
        out_specs=pl.BlockSpec((T, D), lambda i, ept, nr: (i, 0)),
    )
    return pl.pallas_call(
        _ffn_body,
        grid_spec=grid_spec,
        out_shape=jax.ShapeDtypeStruct((Amax, D), jnp.float32),
        compiler_params=pltpu.CompilerParams(
            dimension_semantics=("arbitrary",),
        ),
    )(ept, nreal, x_sorted, W1, b1.reshape(E, 1, F), W2, b2.reshape(E, 1, D),
      scores)


def _make_combine(N, D, Amax):
    mesh = plsc.VectorSubcoreMesh(core_axis_name="c", subcore_axis_name="s")
    tw = N // _NW
    half_rows = tw // 2

    @functools.partial(
        pl.kernel, mesh=mesh,
        out_type=jax.ShapeDtypeStruct((N, D), jnp.float32),
        scratch_types=[
            pltpu.VMEM((2, tw), jnp.int32),
            pltpu.VMEM((half_rows, D), jnp.float32),
            pltpu.VMEM((half_rows, D), jnp.float32),
            pltpu.SemaphoreType.DMA,
        ],
    )
    def combine(y_hbm, dst_hbm, out_hbm, idxbuf, ybuf0, ybuf1, sem):
        w = lax.axis_index("s") * 2 + lax.axis_index("c")
        nch = _NW // 2
        pltpu.sync_copy(dst_hbm.at[w // 2, w % 2], idxbuf.at[0])
        pltpu.sync_copy(dst_hbm.at[nch + w // 2, w % 2], idxbuf.at[1])
        nd = D // 16
        for half in range(2):
            i0 = idxbuf.at[0, pl.ds(half * half_rows, half_rows)]
            i1 = idxbuf.at[1, pl.ds(half * half_rows, half_rows)]
            pltpu.async_copy(y_hbm.at[i0], ybuf0, sem).wait()
            pltpu.async_copy(y_hbm.at[i1], ybuf1, sem).wait()

            def row(r, carry):
                for d in range(nd):
                    sl = pl.ds(d * 16, 16)
                    ybuf0[r, sl] = ybuf0[r, sl] + ybuf1[r, sl]
                return carry

            lax.fori_loop(0, half_rows, row, 0)
            pltpu.sync_copy(
                ybuf0,
                out_hbm.at[pl.ds(w * tw + half * half_rows, half_rows)])

    return combine


def kernel(x, gate_W, gate_b, W1, b1, W2, b2):
    B, L, D = x.shape
    E = gate_W.shape[1]
    N = B * L
    A = 2 * N
    T = _T
    Amax = A + 8 * T
    nt = Amax // T

    xf = x.reshape(N, D)
    w01, e01, base, meta, w01w = _gate(xf, gate_W, gate_b, T, nt)
    ept = meta[0, :nt]
    nreal = meta[0, 40:41]

    x_sorted, dst, scores = _make_dispatch(N, D, E, Amax)(
        e01.reshape(A), base, xf, w01w)

    y_sorted = _grouped_ffn(x_sorted, ept, nreal, W1, b1, W2, b2, scores, T)

    out = _make_combine(N, D, Amax)(y_sorted, dst)
    return out.reshape(B, L, D)


# speedup vs baseline: 1.4742x; 1.0432x over previous
import functools

import jax
import jax.numpy as jnp
from jax import lax
from jax.experimental import pallas as pl
from jax.experimental.pallas import tpu as pltpu
from jax.experimental.pallas import tpu_sc as plsc

_T = 512
_CA = 256
_NW = 32


def _gate_body(nE, T, nt, x_ref, gw_ref, gb_ref, w01_ref, e01_ref, base_ref,
               meta_ref, w01w_ref):
    x = x_ref[...]
    N = x.shape[0]
    logits = jax.lax.dot(x, gw_ref[...],
                         preferred_element_type=jnp.float32) + gb_ref[...]
    p = jax.nn.softmax(logits, axis=-1)
    idx = jax.lax.broadcasted_iota(jnp.int32, p.shape, 1)
    m0 = jnp.max(p, axis=-1, keepdims=True)
    e0 = jnp.min(jnp.where(p == m0, idx, nE), axis=-1, keepdims=True)
    p2 = jnp.where(idx == e0, -jnp.inf, p)
    m1 = jnp.max(p2, axis=-1, keepdims=True)
    e1 = jnp.min(jnp.where(p2 == m1, idx, nE), axis=-1, keepdims=True)
    w01_ref[0, :] = m0[:, 0]
    w01_ref[1, :] = m1[:, 0]
    e01_ref[0, :] = e0[:, 0]
    e01_ref[1, :] = e1[:, 0]
    w01w_ref[...] = jnp.concatenate(
        [jnp.broadcast_to(m0, (N, 128)), jnp.broadcast_to(m1, (N, 128))],
        axis=0)

    oh0 = (idx == e0).astype(jnp.float32)
    oh1 = (idx == e1).astype(jnp.float32)
    nch = N // _CA
    ci = jax.lax.broadcasted_iota(jnp.int32, (nch, N), 0)
    tj = jax.lax.broadcasted_iota(jnp.int32, (nch, N), 1) // _CA
    chunk_ind = (ci == tj).astype(jnp.float32)
    cnt0 = jax.lax.dot(chunk_ind, oh0, preferred_element_type=jnp.float32)
    cnt1 = jax.lax.dot(chunk_ind, oh1, preferred_element_type=jnp.float32)
    cnt = jnp.concatenate([cnt0, cnt1], axis=0)
    tot = jnp.sum(cnt, axis=0, keepdims=True)
    padded = jnp.floor((tot + (T - 1)) * (1.0 / T)) * T
    ri = jax.lax.broadcasted_iota(jnp.int32, (2 * nch, 2 * nch), 0)
    rj = jax.lax.broadcasted_iota(jnp.int32, (2 * nch, 2 * nch), 1)
    tril = (ri > rj).astype(jnp.float32)
    prefix = jax.lax.dot(tril, cnt, preferred_element_type=jnp.float32)
    ui = jax.lax.broadcasted_iota(jnp.int32, (nE, nE), 0)
    uj = jax.lax.broadcasted_iota(jnp.int32, (nE, nE), 1)
    sel = (ui < uj).astype(jnp.float32)
    seg_start = jax.lax.dot(padded, sel, preferred_element_type=jnp.float32)
    base = seg_start + prefix
    base_ref[...] = jnp.concatenate(
        [base.astype(jnp.int32),
         jnp.zeros((2 * nch, 16 - nE), jnp.int32)], axis=1)
    ends = seg_start + padded
    lane = jax.lax.broadcasted_iota(jnp.int32, (1, 64), 1)
    ts = (lane * T).astype(jnp.float32)
    acc = jnp.zeros((1, 64), jnp.float32)
    for e in range(nE):
        acc = acc + (ts >= ends[:, e:e + 1]).astype(jnp.float32)
    ept = jnp.minimum(acc, nE - 1)
    nreal = ends[:, nE - 1:nE] * (1.0 / T)
    meta = jnp.where(lane == 40, nreal, jnp.where(lane < nt, ept, 0.0))
    meta_ref[...] = meta.astype(jnp.int32)


def _gate(xf, gate_W, gate_b, T, nt):
    N, D = xf.shape
    E = gate_W.shape[1]
    return pl.pallas_call(
        functools.partial(_gate_body, E, T, nt),
        in_specs=[
            pl.BlockSpec((N, D), lambda: (0, 0)),
            pl.BlockSpec((D, E), lambda: (0, 0)),
            pl.BlockSpec((1, E), lambda: (0, 0)),
        ],
        out_specs=[
            pl.BlockSpec((2, N), lambda: (0, 0)),
            pl.BlockSpec((2, N), lambda: (0, 0)),
            pl.BlockSpec((2 * (N // _CA), 16), lambda: (0, 0)),
            pl.BlockSpec((1, 64), lambda: (0, 0)),
            pl.BlockSpec((2 * N, 128), lambda: (0, 0)),
        ],
        out_shape=[
            jax.ShapeDtypeStruct((2, N), jnp.float32),
            jax.ShapeDtypeStruct((2, N), jnp.int32),
            jax.ShapeDtypeStruct((2 * (N // _CA), 16), jnp.int32),
            jax.ShapeDtypeStruct((1, 64), jnp.int32),
            jax.ShapeDtypeStruct((2 * N, 128), jnp.float32),
        ],
    )(xf, gate_W, gate_b.reshape(1, E))


def _lane_gather(vec, idx):
    dnums = lax.GatherDimensionNumbers(
        offset_dims=(), collapsed_slice_dims=(0,), start_index_map=(0,))
    return lax.gather(vec, idx[:, None], dnums, slice_sizes=(1,),
                      mode=lax.GatherScatterMode.PROMISE_IN_BOUNDS)


def _make_dispatch(N, D, E, Amax):
    mesh = plsc.VectorSubcoreMesh(core_axis_name="c", subcore_axis_name="s")

    @functools.partial(
        pl.kernel, mesh=mesh,
        out_type=[
            jax.ShapeDtypeStruct((Amax, D), jnp.float32),
            jax.ShapeDtypeStruct((_NW, 2, _CA // 2), jnp.int32),
            jax.ShapeDtypeStruct((Amax, 128), jnp.float32),
        ],
        scratch_types=[
            pltpu.VMEM((_CA,), jnp.int32),
            pltpu.VMEM((16,), jnp.int32),
            pltpu.VMEM((2, _CA // 2), jnp.int32),
            pltpu.VMEM((_CA // 2, D), jnp.float32),
            pltpu.VMEM((_CA // 2, 128), jnp.float32),
            pltpu.SemaphoreType.DMA,
        ],
    )
    def dispatch(e01f_hbm, base_hbm, xf_hbm, w01w_hbm, xs_hbm, dst_hbm,
                 sc_hbm, ebuf, runbuf, dstbuf, xbuf, swbuf, sem):
        w = lax.axis_index("s") * 2 + lax.axis_index("c")
        tok0 = (w % 16) * _CA
        half_rows = _CA // 2
        pltpu.sync_copy(e01f_hbm.at[pl.ds(w * _CA, _CA)], ebuf)
        pltpu.sync_copy(base_hbm.at[w], runbuf)

        iota16 = jax.lax.iota(jnp.int32, 16)
        lane15 = jnp.full((16,), 15, jnp.int32)
        shift_idx = [jnp.maximum(iota16 - (1 << s), 0) for s in range(4)]
        run = runbuf[...]
        for v in range(_CA // 16):
            ae = ebuf[pl.ds(v * 16, 16)]
            dstv = jnp.zeros((16,), jnp.int32)
            for e in range(E):
                mask = ae == e
                csum = jnp.where(mask, 1, 0)
                for s in range(4):
                    moved = _lane_gather(csum, shift_idx[s])
                    csum = csum + jnp.where(iota16 >= (1 << s), moved, 0)
                base_e = _lane_gather(run, jnp.full((16,), e, jnp.int32))
                dstv = jnp.where(mask, base_e + csum - 1, dstv)
                tot = _lane_gather(csum, lane15)
                run = run + jnp.where(iota16 == e, tot, 0)
            dstbuf[v // 8, pl.ds((v % 8) * 16, 16)] = dstv
        pltpu.sync_copy(dstbuf, dst_hbm.at[w])
        for half in range(2):
            pltpu.sync_copy(
                xf_hbm.at[pl.ds(tok0 + half * half_rows, half_rows)], xbuf)
            pltpu.async_copy(xbuf, xs_hbm.at[dstbuf.at[half]], sem).wait()
            pltpu.sync_copy(
                w01w_hbm.at[pl.ds(w * _CA + half * half_rows, half_rows)],
                swbuf)
            pltpu.async_copy(swbuf, sc_hbm.at[dstbuf.at[half]], sem).wait()

    return dispatch


def _ffn_body(ept_ref, nreal_ref, x_ref, w1_ref, b1_ref, w2_ref, b2_ref,
              s_ref, y_ref):
    i = pl.program_id(0)

    @pl.when(i < nreal_ref[0])
    def _():
        x = x_ref[...]
        h = jax.lax.dot(x, w1_ref[0], preferred_element_type=jnp.float32)
        h = h + b1_ref[0]
        h = 0.5 * h * (1.0 + jax.lax.erf(h * (2.0 ** -0.5)))
        y = jax.lax.dot(h, w2_ref[0], preferred_element_type=jnp.float32)
        scale = jnp.max(s_ref[...], axis=1, keepdims=True)
        y_ref[...] = (y + b2_ref[0]) * scale


def _grouped_ffn(x_sorted, ept, nreal, W1, b1, W2, b2, scores, T):
    Amax, D = x_sorted.shape
    E, _, F = W1.shape
    nt = Amax // T
    grid_spec = pltpu.PrefetchScalarGridSpec(
        num_scalar_prefetch=2,
        grid=(nt,),
        in_specs=[
            pl.BlockSpec((T, D), lambda i, ept, nr: (i, 0)),
            pl.BlockSpec((1, D, F), lambda i, ept, nr: (ept[i], 0, 0)),
            pl.BlockSpec((1, 1, F), lambda i, ept, nr: (ept[i], 0, 0)),
            pl.BlockSpec((1, F, D), lambda i, ept, nr: (ept[i], 0, 0)),
            pl.BlockSpec((1, 1, D), lambda i, ept, nr: (ept[i], 0, 0)),
            pl.BlockSpec((T, 128), lambda i, ept, nr: (i, 0)),
        ],
        out_specs=pl.BlockSpec((T, D), lambda i, ept, nr: (i, 0)),
    )
    return pl.pallas_call(
        _ffn_body,
        grid_spec=grid_spec,
        out_shape=jax.ShapeDtypeStruct((Amax, D), jnp.float32),
        compiler_params=pltpu.CompilerParams(
            dimension_semantics=("arbitrary",),
        ),
    )(ept, nreal, x_sorted, W1, b1.reshape(E, 1, F), W2, b2.reshape(E, 1, D),
      scores)


def _make_combine(N, D, Amax):
    mesh = plsc.VectorSubcoreMesh(core_axis_name="c", subcore_axis_name="s")
    tw = N // _NW
    half_rows = tw // 2

    @functools.partial(
        pl.kernel, mesh=mesh,
        out_type=jax.ShapeDtypeStruct((N, D), jnp.float32),
        scratch_types=[
            pltpu.VMEM((2, tw), jnp.int32),
            pltpu.VMEM((half_rows, D), jnp.float32),
            pltpu.VMEM((half_rows, D), jnp.float32),
            pltpu.SemaphoreType.DMA,
        ],
    )
    def combine(y_hbm, dst_hbm, out_hbm, idxbuf, ybuf0, ybuf1, sem):
        w = lax.axis_index("s") * 2 + lax.axis_index("c")
        nch = _NW // 2
        pltpu.sync_copy(dst_hbm.at[w // 2, w % 2], idxbuf.at[0])
        pltpu.sync_copy(dst_hbm.at[nch + w // 2, w % 2], idxbuf.at[1])
        nd = D // 16
        for half in range(2):
            i0 = idxbuf.at[0, pl.ds(half * half_rows, half_rows)]
            i1 = idxbuf.at[1, pl.ds(half * half_rows, half_rows)]
            pltpu.async_copy(y_hbm.at[i0], ybuf0, sem).wait()
            pltpu.async_copy(y_hbm.at[i1], ybuf1, sem).wait()

            def row(r, carry):
                for d in range(nd):
                    sl = pl.ds(d * 16, 16)
                    ybuf0[r, sl] = ybuf0[r, sl] + ybuf1[r, sl]
                return carry

            lax.fori_loop(0, half_rows, row, 0)
            pltpu.sync_copy(
                ybuf0,
                out_hbm.at[pl.ds(w * tw + half * half_rows, half_rows)])

    return combine


def kernel(x, gate_W, gate_b, W1, b1, W2, b2):
    B, L, D = x.shape
    E = gate_W.shape[1]
    N = B * L
    A = 2 * N
    T = _T
    Amax = A + 8 * T
    nt = Amax // T

    xf = x.reshape(N, D)
    w01, e01, base, meta, w01w = _gate(xf, gate_W, gate_b, T, nt)
    ept = meta[0, :nt]
    nreal = meta[0, 40:41]

    x_sorted, dst, scores = _make_dispatch(N, D, E, Amax)(
        e01.reshape(A), base, xf, w01w)

    y_sorted = _grouped_ffn(x_sorted, ept, nreal, W1, b1, W2, b2, scores, T)

    out = _make_combine(N, D, Amax)(y_sorted, dst)
    return out.reshape(B, L, D)
